# baseline (speedup 1.0000x reference)
import functools

import jax
import jax.numpy as jnp
from jax import lax
from jax.experimental import pallas as pl
from jax.experimental.pallas import tpu as pltpu
from jax.experimental.pallas import tpu_sc as plsc

NC = 2
NS = 16
NW = NC * NS


CH = 64


def _sc_aggregate(h, cmb, w, zeros, nch, ch):
    n, d = h.shape
    rpt = (n // NS) & ~7
    tail = n - NS * rpt
    ib = 2 * ch
    epw = nch * ch

    mesh = plsc.VectorSubcoreMesh(core_axis_name="c", subcore_axis_name="s")

    @functools.partial(
        pl.kernel,
        out_type=jax.ShapeDtypeStruct((NC, n, d), jnp.float32),
        mesh=mesh,
        scratch_types=[
            pltpu.VMEM((ib,), jnp.int32),
            pltpu.VMEM((ib,), jnp.int32),
            pltpu.VMEM((ib,), jnp.int32),
            pltpu.VMEM((ib,), jnp.int32),
            pltpu.VMEM((epw,), jnp.float32),
            pltpu.VMEM((ch, d), jnp.float32),
            pltpu.VMEM((ch, d), jnp.float32),
            pltpu.VMEM((ch, d), jnp.float32),
            pltpu.VMEM((ch, d), jnp.float32),
            pltpu.VMEM_SHARED((n, d), jnp.float32),
            pltpu.SemaphoreType.DMA,
            pltpu.SemaphoreType.DMA,
            pltpu.SemaphoreType.DMA,
            pltpu.SemaphoreType.DMA,
            pltpu.SemaphoreType.DMA,
            pltpu.SemaphoreType.DMA,
            pltpu.SemaphoreType.DMA,
            pltpu.SemaphoreType.DMA,
        ],
    )
    def k(h_hbm, cmb_hbm, w_hbm, z_hbm, out_hbm,
          ib0, ib1, ib2, ib3, wv, rb0, rb1, sb0, sb1, acc,
          is0, is1, is2, is3, rs0, rs1, ss0, ss1):
        cid = lax.axis_index("c")
        sid = lax.axis_index("s")
        wid = sid * NC + cid
        cbase = wid * nch
        ibufs = (ib0, ib1, ib2, ib3)
        isems = (is0, is1, is2, is3)
        rbufs = (rb0, rb1)
        rsems = (rs0, rs1)
        sbufs = (sb0, sb1)
        ssems = (ss0, ss1)

        def issue_idx(c, slot):
            off = pl.multiple_of((cbase + c) * ib, 8)
            pltpu.async_copy(cmb_hbm.at[pl.ds(off, ib)], ibufs[slot],
                             isems[slot])

        def wait_idx(slot):
            pltpu.make_async_copy(cmb_hbm.at[pl.ds(0, ib)], ibufs[slot],
                                  isems[slot]).wait()

        def issue_rows(slot, b2):
            pltpu.async_copy(h_hbm.at[ibufs[slot].at[pl.ds(0, ch)]],
                             rbufs[b2], rsems[b2])

        def wait_rows(b2):
            pltpu.make_async_copy(h_hbm.at[pl.ds(0, ch)], rbufs[b2],
                                  rsems[b2]).wait()

        def issue_scatter(slot, b2):
            pltpu.async_copy(sbufs[b2], acc.at[ibufs[slot].at[pl.ds(ch, ch)]],
                             ssems[b2], add=True)

        def wait_scatter(b2):
            pltpu.make_async_copy(sbufs[b2], acc.at[pl.ds(0, ch)],
                                  ssems[b2]).wait()

        woff = pl.multiple_of(wid * epw, 8)
        pltpu.sync_copy(w_hbm.at[pl.ds(woff, epw)], wv)
        r0 = pl.multiple_of(sid * rpt, 8)
        pltpu.sync_copy(z_hbm.at[pl.ds(0, rpt)], acc.at[pl.ds(r0, rpt)])
        if tail:
            @pl.when(sid == NS - 1)
            def _():
                pltpu.sync_copy(z_hbm.at[pl.ds(0, tail)],
                                acc.at[pl.ds(NS * rpt, tail)])

        for r in range(min(4, nch)):
            issue_idx(r, r)
        for c0 in range(min(2, nch)):
            wait_idx(c0)
            issue_rows(c0, c0)
        plsc.subcore_barrier()

        def scale(c, b2):
            grows, srows = rbufs[b2], sbufs[b2]
            base = c * ch
            for g0 in range(0, ch, 16):
                wvec = wv[pl.ds(base + g0, 16)]
                for j in range(16):
                    ei = g0 + j
                    we = wvec[j]
                    for kk in range(d // 16):
                        sl = pl.ds(kk * 16, 16)
                        srows[ei, sl] = grows[ei, sl] * we

        def process(c, slot, b2):
            stat = isinstance(c, int)
            nslot = (slot + 2) % 4
            if stat:
                if c >= 2:
                    wait_scatter(b2)
            else:
                @pl.when(c >= 2)
                def _():
                    wait_scatter(b2)
            if stat:
                if c >= 2 and c + 2 < nch:
                    issue_idx(c + 2, nslot)
            else:
                @pl.when(jnp.logical_and(c >= 2, c + 2 < nch))
                def _():
                    issue_idx(c + 2, nslot)
            wait_rows(b2)
            scale(c, b2)
            if stat:
                if c + 2 < nch:
                    wait_idx(nslot)
                    issue_rows(nslot, b2)
            else:
                @pl.when(c + 2 < nch)
                def _():
                    wait_idx(nslot)
                    issue_rows(nslot, b2)
            issue_scatter(slot, b2)

        nquad = nch // 4

        def body(i, carry):
            c0 = i * 4
            for q in range(4):
                process(c0 + q, q, q % 2)
            return carry

        lax.fori_loop(0, nquad, body, 0)
        for c in range(nquad * 4, nch):
            process(c, c % 4, c % 2)
        for b2 in range(min(2, nch)):
            wait_scatter(b2)
        plsc.subcore_barrier()
        pltpu.sync_copy(acc.at[pl.ds(r0, rpt)], out_hbm.at[cid, pl.ds(r0, rpt)])
        if tail:
            @pl.when(sid == NS - 1)
            def _():
                pltpu.sync_copy(acc.at[pl.ds(NS * rpt, tail)],
                                out_hbm.at[cid, pl.ds(NS * rpt, tail)])

    return k(h, cmb, w, zeros)


def _tc_matmul(x, w):
    m, kdim = x.shape
    nout = w.shape[1]
    nb = 10
    bm = m // nb

    def body(x_ref, w_ref, o_ref):
        o_ref[...] = jnp.dot(x_ref[...], w_ref[...],
                             preferred_element_type=jnp.float32)

    return pl.pallas_call(
        body,
        grid=(nb,),
        in_specs=[pl.BlockSpec((bm, kdim), lambda i: (i, 0)),
                  pl.BlockSpec((kdim, nout), lambda i: (0, 0))],
        out_specs=pl.BlockSpec((bm, nout), lambda i: (i, 0)),
        out_shape=jax.ShapeDtypeStruct((m, nout), jnp.float32),
    )(x, w)


def _tc_combine_relu_matmul(parts, b, w):
    _, m, hdim = parts.shape
    nout = w.shape[1]
    nb = 10
    bm = m // nb
    b2d = b.reshape(1, hdim)

    def body(p_ref, b_ref, w_ref, o_ref):
        x = jnp.maximum(p_ref[0] + p_ref[1] + b_ref[...], 0.0)
        o_ref[...] = jnp.dot(x, w_ref[...], preferred_element_type=jnp.float32)

    return pl.pallas_call(
        body,
        grid=(nb,),
        in_specs=[pl.BlockSpec((NC, bm, hdim), lambda i: (0, i, 0)),
                  pl.BlockSpec((1, hdim), lambda i: (0, 0)),
                  pl.BlockSpec((hdim, nout), lambda i: (0, 0))],
        out_specs=pl.BlockSpec((bm, nout), lambda i: (i, 0)),
        out_shape=jax.ShapeDtypeStruct((m, nout), jnp.float32),
    )(parts, b2d, w)


def _tc_combine_bias(parts, b):
    _, m, hdim = parts.shape
    nb = 10
    bm = m // nb
    b2d = b.reshape(1, hdim)

    def body(p_ref, b_ref, o_ref):
        o_ref[...] = p_ref[0] + p_ref[1] + b_ref[...]

    return pl.pallas_call(
        body,
        grid=(nb,),
        in_specs=[pl.BlockSpec((NC, bm, hdim), lambda i: (0, i, 0)),
                  pl.BlockSpec((1, hdim), lambda i: (0, 0))],
        out_specs=pl.BlockSpec((bm, hdim), lambda i: (i, 0)),
        out_shape=jax.ShapeDtypeStruct((m, hdim), jnp.float32),
    )(parts, b2d)


def kernel(edge_index, edge_weight, emb, W1, b1, W2, b2):
    src = edge_index[0]
    dst = edge_index[1]
    n, d = emb.shape
    e = edge_weight.shape[0]
    epw = e // NW
    ch = CH
    nch = -(-epw // ch)
    pad = nch * ch - epw
    src2 = jnp.pad(src.reshape(NW, epw), ((0, 0), (0, pad)))
    dst2 = jnp.pad(dst.reshape(NW, epw), ((0, 0), (0, pad)))
    wpad = jnp.pad(edge_weight.reshape(NW, epw), ((0, 0), (0, pad))).reshape(-1)
    cmb = jnp.stack([src2.reshape(NW, nch, ch),
                     dst2.reshape(NW, nch, ch)], axis=2).reshape(-1)
    zeros = jnp.zeros(((n // NS) & ~7, d), dtype=jnp.float32)

    h1 = _tc_matmul(emb, W1)
    parts1 = _sc_aggregate(h1, cmb, wpad, zeros, nch, ch)
    h2 = _tc_combine_relu_matmul(parts1, b1, W2)
    parts2 = _sc_aggregate(h2, cmb, wpad, zeros, nch, ch)
    return _tc_combine_bias(parts2, b2)


# speedup vs baseline: 1.4176x; 1.4176x over previous
import functools

import jax
import jax.numpy as jnp
from jax import lax
from jax.experimental import pallas as pl
from jax.experimental.pallas import tpu as pltpu
from jax.experimental.pallas import tpu_sc as plsc

NC = 2
NS = 16
NW = NC * NS


CH = 64


def _sc_aggregate(h, src, dst, w, zeros, mode=3):
    n, d = h.shape
    e = src.shape[0]
    epw = e // NW
    ch = 80
    nch = epw // ch
    rpt = (n // NS) & ~7
    tail = n - NS * rpt

    mesh = plsc.VectorSubcoreMesh(core_axis_name="c", subcore_axis_name="s")

    @functools.partial(
        pl.kernel,
        out_type=jax.ShapeDtypeStruct((NC, n, d), jnp.float32),
        mesh=mesh,
        scratch_types=[
            pltpu.VMEM((epw,), jnp.int32),
            pltpu.VMEM((epw,), jnp.int32),
            pltpu.VMEM((epw + 16,), jnp.float32),
            pltpu.VMEM((ch, d), jnp.float32),
            pltpu.VMEM((ch, d), jnp.float32),
            pltpu.VMEM_SHARED((n, d), jnp.float32),
            pltpu.SemaphoreType.DMA,
            pltpu.SemaphoreType.DMA,
        ],
    )
    def k(h_hbm, src_hbm, dst_hbm, w_hbm, z_hbm, out_hbm,
          srcv, dstv, wv, rows_a, rows_b, acc, sem_a, sem_b):
        cid = lax.axis_index("c")
        sid = lax.axis_index("s")
        wid = sid * NC + cid
        woff = pl.multiple_of(wid * epw, 8)
        pltpu.sync_copy(src_hbm.at[pl.ds(woff, epw)], srcv)
        pltpu.sync_copy(dst_hbm.at[pl.ds(woff, epw)], dstv)
        pltpu.sync_copy(w_hbm.at[pl.ds(woff, epw)], wv.at[pl.ds(0, epw)])
        r0 = pl.multiple_of(sid * rpt, 8)
        pltpu.sync_copy(z_hbm.at[pl.ds(0, rpt)], acc.at[pl.ds(r0, rpt)])
        if tail:
            @pl.when(sid == NS - 1)
            def _():
                pltpu.sync_copy(z_hbm.at[pl.ds(0, tail)],
                                acc.at[pl.ds(NS * rpt, tail)])
        bufs = (rows_a, rows_b)
        sems = (sem_a, sem_b)
        pltpu.async_copy(h_hbm.at[srcv.at[pl.ds(0, ch)]], rows_a, sem_a)
        if nch > 1:
            pltpu.async_copy(h_hbm.at[srcv.at[pl.ds(ch, ch)]], rows_b, sem_b)
        plsc.subcore_barrier()

        def scale_and_scatter(c, rows, sem):
            pltpu.make_async_copy(h_hbm.at[pl.ds(0, ch)], rows, sem).wait()
            base = c * ch
            if mode & 1:
                for g0 in range(0, ch, 16):
                    wvec = wv[pl.ds(base + g0, 16)]
                    for j in range(min(16, ch - g0)):
                        ei = g0 + j
                        we = wvec[j]
                        for kk in range(d // 16):
                            sl = pl.ds(kk * 16, 16)
                            rows[ei, sl] = rows[ei, sl] * we
            if mode & 2:
                pltpu.sync_copy(rows, acc.at[dstv.at[pl.ds(base, ch)]],
                                add=True)

        def body(i, carry):
            for b in range(2):
                c = 2 * i + b
                scale_and_scatter(c, bufs[b], sems[b])
                nxt = c + 2

                @pl.when(nxt < nch)
                def _():
                    pltpu.async_copy(h_hbm.at[srcv.at[pl.ds(nxt * ch, ch)]],
                                     bufs[b], sems[b])
            return carry

        lax.fori_loop(0, nch // 2, body, 0)
        if nch % 2:
            scale_and_scatter(nch - 1, bufs[(nch - 1) % 2], sems[(nch - 1) % 2])
        plsc.subcore_barrier()
        pltpu.sync_copy(acc.at[pl.ds(r0, rpt)], out_hbm.at[cid, pl.ds(r0, rpt)])
        if tail:
            @pl.when(sid == NS - 1)
            def _():
                pltpu.sync_copy(acc.at[pl.ds(NS * rpt, tail)],
                                out_hbm.at[cid, pl.ds(NS * rpt, tail)])

    return k(h, src, dst, w, zeros)


def _tc_matmul(x, w):
    m, kdim = x.shape
    nout = w.shape[1]
    nb = 10
    bm = m // nb

    def body(x_ref, w_ref, o_ref):
        o_ref[...] = jnp.dot(x_ref[...], w_ref[...],
                             preferred_element_type=jnp.float32)

    return pl.pallas_call(
        body,
        grid=(nb,),
        in_specs=[pl.BlockSpec((bm, kdim), lambda i: (i, 0)),
                  pl.BlockSpec((kdim, nout), lambda i: (0, 0))],
        out_specs=pl.BlockSpec((bm, nout), lambda i: (i, 0)),
        out_shape=jax.ShapeDtypeStruct((m, nout), jnp.float32),
    )(x, w)


def _tc_combine_relu_matmul(parts, b, w):
    _, m, hdim = parts.shape
    nout = w.shape[1]
    nb = 10
    bm = m // nb
    b2d = b.reshape(1, hdim)

    def body(p_ref, b_ref, w_ref, o_ref):
        x = jnp.maximum(p_ref[0] + p_ref[1] + b_ref[...], 0.0)
        o_ref[...] = jnp.dot(x, w_ref[...], preferred_element_type=jnp.float32)

    return pl.pallas_call(
        body,
        grid=(nb,),
        in_specs=[pl.BlockSpec((NC, bm, hdim), lambda i: (0, i, 0)),
                  pl.BlockSpec((1, hdim), lambda i: (0, 0)),
                  pl.BlockSpec((hdim, nout), lambda i: (0, 0))],
        out_specs=pl.BlockSpec((bm, nout), lambda i: (i, 0)),
        out_shape=jax.ShapeDtypeStruct((m, nout), jnp.float32),
    )(parts, b2d, w)


def _tc_combine_bias(parts, b):
    _, m, hdim = parts.shape
    nb = 10
    bm = m // nb
    b2d = b.reshape(1, hdim)

    def body(p_ref, b_ref, o_ref):
        o_ref[...] = p_ref[0] + p_ref[1] + b_ref[...]

    return pl.pallas_call(
        body,
        grid=(nb,),
        in_specs=[pl.BlockSpec((NC, bm, hdim), lambda i: (0, i, 0)),
                  pl.BlockSpec((1, hdim), lambda i: (0, 0))],
        out_specs=pl.BlockSpec((bm, hdim), lambda i: (i, 0)),
        out_shape=jax.ShapeDtypeStruct((m, hdim), jnp.float32),
    )(parts, b2d)


def kernel(edge_index, edge_weight, emb, W1, b1, W2, b2):
    src = edge_index[0]
    dst = edge_index[1]
    n, d = emb.shape
    zeros = jnp.zeros(((n // NS) & ~7, d), dtype=jnp.float32)

    h1 = _tc_matmul(emb, W1)
    parts1 = _sc_aggregate(h1, src, dst, edge_weight, zeros)
    h2 = _tc_combine_relu_matmul(parts1, b1, W2)
    parts2 = _sc_aggregate(h2, src, dst, edge_weight, zeros)
    return _tc_combine_bias(parts2, b2)
